# baseline (speedup 1.0000x reference)
import functools

import jax
import jax.numpy as jnp
from jax import lax
from jax.experimental import pallas as pl
from jax.experimental.pallas import tpu as pltpu
from jax.experimental.pallas import tpu_sc as plsc

N = 10000
E = 320000
D = 128
G = 64

NC, NS = 2, 16
NW = NC * NS
NP = 10240
EB = 128
EP = 327680
EPW = EP // NW
NBLK = EPW // EB
HALF = NBLK // 2
RPT = NP // NS

_HIGH = lax.Precision.HIGHEST


def _edge_body(u_hbm, src_hbm, dst_hbm, zeros_hbm, out_hbm,
               sidx, didx, rows0, rows1, acc, sem, gsem0, gsem1):
    c = lax.axis_index("c")
    s = lax.axis_index("s")
    w = c * NS + s

    for k in range(RPT // EB):
        pltpu.async_copy(zeros_hbm, acc.at[pl.ds(s * RPT + k * EB, EB)], sem)
    pltpu.async_copy(src_hbm.at[w, 0], sidx, sem)
    pltpu.async_copy(dst_hbm.at[w, 0], didx, sem)
    for k in range(RPT // EB):
        pltpu.make_async_copy(zeros_hbm, acc.at[pl.ds(s * RPT + k * EB, EB)],
                              sem).wait()
    pltpu.make_async_copy(src_hbm.at[w, 0], sidx, sem).wait()
    pltpu.make_async_copy(dst_hbm.at[w, 0], didx, sem).wait()
    plsc.subcore_barrier()

    def _gather(j, rows, gsem):
        pltpu.async_copy(u_hbm.at[sidx.at[j]], rows, gsem)

    def _gwait(j, rows, gsem):
        pltpu.make_async_copy(u_hbm.at[sidx.at[j]], rows, gsem).wait()

    def _scatter(j, rows):
        pltpu.sync_copy(rows, acc.at[didx.at[j]], add=True)

    for h in range(2):
        _gather(0, rows0, gsem0)

        @pl.loop(0, HALF - 2, step=2)
        def _(j):
            _gwait(j, rows0, gsem0)
            _gather(j + 1, rows1, gsem1)
            _scatter(j, rows0)
            _gwait(j + 1, rows1, gsem1)
            _gather(j + 2, rows0, gsem0)
            _scatter(j + 1, rows1)

        _gwait(HALF - 2, rows0, gsem0)
        _gather(HALF - 1, rows1, gsem1)
        _scatter(HALF - 2, rows0)
        _gwait(HALF - 1, rows1, gsem1)
        _scatter(HALF - 1, rows1)
        if h == 0:
            pltpu.sync_copy(src_hbm.at[w, 1], sidx)
            pltpu.sync_copy(dst_hbm.at[w, 1], didx)

    plsc.subcore_barrier()
    pltpu.sync_copy(acc.at[pl.ds(s * RPT, RPT)], out_hbm.at[c, pl.ds(s * RPT, RPT)])


@functools.cache
def _sc_kernels():
    mesh = plsc.VectorSubcoreMesh(
        core_axis_name="c", subcore_axis_name="s",
        num_cores=NC, num_subcores=NS,
    )
    edge = pl.kernel(
        _edge_body,
        out_type=jax.ShapeDtypeStruct((NC, NP, D), jnp.float32),
        mesh=mesh,
        scratch_types=[
            pltpu.VMEM((HALF, EB), jnp.int32),
            pltpu.VMEM((HALF, EB), jnp.int32),
            pltpu.VMEM((EB, D), jnp.float32),
            pltpu.VMEM((EB, D), jnp.float32),
            pltpu.VMEM_SHARED((NP, D), jnp.float32),
            pltpu.SemaphoreType.DMA,
            pltpu.SemaphoreType.DMA,
            pltpu.SemaphoreType.DMA,
        ],
    )
    return edge


_R = 1024
_NG = NP // _R


def _dinv_of(deg_ref):
    deg = deg_ref[0, :, 0:1] + deg_ref[1, :, 0:1] + 1.0
    return lax.rsqrt(deg)


def _first_body(deg_ref, x_ref, w_ref, o_ref):
    dinv = _dinv_of(deg_ref)
    h = jnp.dot(x_ref[...], w_ref[...], precision=_HIGH,
                preferred_element_type=jnp.float32)
    o_ref[...] = h * dinv


def _fused_body(deg_ref, p_ref, u_ref, b_ref, w_ref, o_ref):
    dinv = _dinv_of(deg_ref)
    agg = dinv * (p_ref[0] + p_ref[1] + u_ref[...]) + b_ref[...]
    y = jnp.maximum(agg, 0.0)
    h = jnp.dot(y, w_ref[...], precision=_HIGH,
                preferred_element_type=jnp.float32)
    o_ref[...] = h * dinv


def _final_body(deg_ref, p_ref, u_ref, b_ref, batch_ref, wout_ref, bout_ref,
                o_ref, sums, cnts):
    i = pl.program_id(0)
    dinv = _dinv_of(deg_ref)
    agg = dinv * (p_ref[0] + p_ref[1] + u_ref[...]) + b_ref[...]
    y = jnp.maximum(agg, 0.0)
    oh = (batch_ref[...] == lax.broadcasted_iota(jnp.int32, (_R, G), 1))
    oh = oh.astype(jnp.float32)
    part = lax.dot_general(oh, y, (((0,), (0,)), ((), ())), precision=_HIGH,
                           preferred_element_type=jnp.float32)
    cpart = jnp.sum(oh, axis=0)[:, None]

    @pl.when(i == 0)
    def _():
        sums[...] = jnp.zeros_like(sums)
        cnts[...] = jnp.zeros_like(cnts)

    sums[...] += part
    cnts[...] += cpart

    @pl.when(i == _NG - 1)
    def _():
        pooled = sums[...] / jnp.maximum(cnts[...], 1.0)
        o_ref[...] = (
            jnp.dot(pooled, wout_ref[...], precision=_HIGH,
                    preferred_element_type=jnp.float32)
            + bout_ref[...]
        )


_deg_spec = pl.BlockSpec((NC, _R, D), lambda i: (0, i, 0))
_row_spec = pl.BlockSpec((_R, D), lambda i: (i, 0))
_p_spec = pl.BlockSpec((NC, _R, D), lambda i: (0, i, 0))
_w_spec = pl.BlockSpec((D, D), lambda i: (0, 0))
_b_spec = pl.BlockSpec((1, D), lambda i: (0, 0))

_first_tc = pl.pallas_call(
    _first_body,
    grid=(_NG,),
    in_specs=[_deg_spec, _row_spec, _w_spec],
    out_specs=_row_spec,
    out_shape=jax.ShapeDtypeStruct((NP, D), jnp.float32),
)

_fused_tc = pl.pallas_call(
    _fused_body,
    grid=(_NG,),
    in_specs=[_deg_spec, _p_spec, _row_spec, _b_spec, _w_spec],
    out_specs=_row_spec,
    out_shape=jax.ShapeDtypeStruct((NP, D), jnp.float32),
)

_final_tc = pl.pallas_call(
    _final_body,
    grid=(_NG,),
    in_specs=[_deg_spec, _p_spec, _row_spec, _b_spec,
              pl.BlockSpec((_R, 1), lambda i: (i, 0)),
              _w_spec, _b_spec],
    out_specs=pl.BlockSpec((G, D), lambda i: (0, 0)),
    out_shape=jax.ShapeDtypeStruct((G, D), jnp.float32),
    scratch_shapes=[pltpu.VMEM((G, D), jnp.float32),
                    pltpu.VMEM((G, 1), jnp.float32)],
)


def kernel(x, edge_index, batch, W1, b1, W2, b2, W3, b3, Wout, bout):
    pad_e = N + jnp.arange(EP - E, dtype=jnp.int32) % (NP - N)
    src = jnp.concatenate([edge_index[0], pad_e]).reshape(NW, 2, HALF, EB)
    dst = jnp.concatenate([edge_index[1], pad_e]).reshape(NW, 2, HALF, EB)
    x_p = jnp.concatenate([x, jnp.zeros((NP - N, D), jnp.float32)], axis=0)
    batch_p = jnp.concatenate(
        [batch, jnp.full((NP - N,), G, dtype=batch.dtype)]
    ).reshape(NP, 1)
    zeros_row = jnp.zeros((EB, D), jnp.float32)
    ones_mat = jnp.ones((NP, D), jnp.float32)
    b1r, b2r, b3r = b1.reshape(1, D), b2.reshape(1, D), b3.reshape(1, D)
    boutr = bout.reshape(1, D)

    edge_k = _sc_kernels()
    degp = edge_k(ones_mat, src, dst, zeros_row)
    u1 = _first_tc(degp, x_p, W1)
    p1 = edge_k(u1, src, dst, zeros_row)
    u2 = _fused_tc(degp, p1, u1, b1r, W2)
    p2 = edge_k(u2, src, dst, zeros_row)
    u3 = _fused_tc(degp, p2, u2, b2r, W3)
    p3 = edge_k(u3, src, dst, zeros_row)
    return _final_tc(degp, p3, u3, b3r, batch_p, Wout, boutr)
